# baseline (speedup 1.0000x reference)
import jax
import jax.numpy as jnp
from jax import lax
from jax.experimental import pallas as pl
from jax.experimental.pallas import tpu as pltpu
from jax.experimental.pallas import tpu_sc as plsc

N = 10000
E = 320000
D = 128
DH = D // 2
K = 5
ALPHA = 0.8

NC = 2
NS = 16
ES = E // NS
CH = 125
NCH = ES // CH
NPAD = 10240
RPS = NPAD // NS
DCH = 128
NDCH = RPS // DCH
NBUF = 5

DEG_CH = 80
DEG_NCH = N // DEG_CH
EW = E // (NC * NS)

BF = jnp.bfloat16


def _deg_body(dst_hbm, zdeg_hbm, ridx_hbm, deg_out_hbm,
              dst_v, deg_l, ridx_v, deg_sp):
    c = lax.axis_index("c")
    s = lax.axis_index("s")
    w = s * NC + c

    pltpu.sync_copy(zdeg_hbm, deg_l)

    @pl.when(s == 0)
    def _():
        pltpu.sync_copy(deg_l, deg_sp)

    plsc.subcore_barrier()

    pltpu.sync_copy(dst_hbm.at[w], dst_v)
    pltpu.sync_copy(ridx_hbm, ridx_v)
    ones = jnp.ones((16,), jnp.float32)

    @pl.loop(0, EW // 16)
    def _(i):
        d = dst_v[pl.ds(i * 16, 16)]
        plsc.addupdate_scatter(deg_l, [d], ones)

    @pl.loop(0, DEG_NCH)
    def _(j):
        pltpu.sync_copy(deg_l.at[pl.ds(j * DEG_CH, DEG_CH)],
                        deg_sp.at[ridx_v.at[j]], add=True)

    plsc.subcore_barrier()

    @pl.when(s == 0)
    def _():
        pltpu.sync_copy(deg_sp, deg_l)
        pltpu.sync_copy(deg_l, deg_out_hbm.at[c])


def _mono_body(xsd0_hbm, xsd1_hbm, af_hbm, sqdb_hbm, src_hbm, dst_hbm,
               zero_hbm,
               oout0_hbm, oout1_hbm, osb0_hbm, osb1_hbm,
               src_v, dst_v, rows0, rows1, rows2, rows3, rows4,
               os_res, xch_v, aggch_v, af_v, agg_sp,
               sg0, sg1, sg2, sg3, sg4):
    rows = (rows0, rows1, rows2, rows3, rows4)
    sg = (sg0, sg1, sg2, sg3, sg4)
    c = lax.axis_index("c")
    s = lax.axis_index("s")
    base = s * RPS

    pltpu.sync_copy(src_hbm.at[s], src_v)
    pltpu.sync_copy(dst_hbm.at[s], dst_v)
    pltpu.sync_copy(af_hbm.at[pl.ds(base, RPS)], af_v)

    def half(xsd_hbm, osb_hbm, oout_hbm):
        @pl.loop(0, NDCH)
        def _(t):
            pltpu.sync_copy(xsd_hbm.at[pl.ds(base + t * DCH, DCH)], xch_v)

            @pl.loop(0, DCH)
            def _(r):
                rr = t * DCH + r
                for g in range(2):
                    sl = pl.ds(32 * g, 32)
                    os_res[rr, sl] = xch_v[r, sl] * jnp.asarray(1.25, BF)

        pltpu.sync_copy(os_res, osb_hbm.at[pl.ds(base, RPS)])

        @pl.loop(0, K)
        def _(k):
            pltpu.sync_copy(zero_hbm, aggch_v)

            @pl.loop(0, NDCH)
            def _(t):
                pltpu.sync_copy(aggch_v, agg_sp.at[pl.ds(base + t * DCH, DCH)])

            plsc.subcore_barrier()

            for b in range(NBUF):
                pltpu.async_copy(osb_hbm.at[src_v.at[b]], rows[b], sg[b])

            @pl.loop(0, NCH // NBUF)
            def _(t):
                for b in range(NBUF):
                    j = t * NBUF + b
                    pltpu.make_async_copy(osb_hbm.at[src_v.at[j]],
                                          rows[b], sg[b]).wait()
                    pltpu.sync_copy(rows[b], agg_sp.at[dst_v.at[j]], add=True)

                    @pl.when(j < NCH - NBUF)
                    def _():
                        pltpu.async_copy(osb_hbm.at[src_v.at[j + NBUF]],
                                         rows[b], sg[b])

            plsc.subcore_barrier()

            @pl.loop(0, NDCH)
            def _(t):
                row0 = base + t * DCH
                pltpu.sync_copy(agg_sp.at[pl.ds(row0, DCH)], aggch_v)
                pltpu.sync_copy(xsd_hbm.at[pl.ds(row0, DCH)], xch_v)

                @pl.loop(0, DCH)
                def _(r):
                    rr = t * DCH + r
                    for g in range(2):
                        sl = pl.ds(32 * g, 32)
                        os_res[rr, sl] = (af_v[rr, sl]
                                          * (aggch_v[r, sl] + os_res[rr, sl])
                                          + xch_v[r, sl])

            pltpu.sync_copy(os_res, osb_hbm.at[pl.ds(base, RPS)])

        @pl.loop(0, NDCH)
        def _(t):
            row0 = base + t * DCH
            pltpu.sync_copy(sqdb_hbm.at[pl.ds(row0, DCH)], aggch_v)

            @pl.loop(0, DCH)
            def _(r):
                rr = t * DCH + r
                for g in range(2):
                    sl = pl.ds(32 * g, 32)
                    os_res[rr, sl] = os_res[rr, sl] * aggch_v[r, sl]

        pltpu.sync_copy(os_res, oout_hbm.at[pl.ds(base, RPS)])

    @pl.when(c == 0)
    def _():
        half(xsd0_hbm, osb0_hbm, oout0_hbm)

    @pl.when(c == 1)
    def _():
        half(xsd1_hbm, osb1_hbm, oout1_hbm)


def _sc_mesh():
    return plsc.VectorSubcoreMesh(core_axis_name="c", subcore_axis_name="s",
                                  num_cores=NC, num_subcores=NS)


_RB = 2000


def _prep_body(degp_ref, x_ref, af_ref, sqdb_ref, xsd0_ref, xsd1_ref):
    p = degp_ref[...]
    deg = p[:, 0:1] + p[:, 1:2] + 1.0
    dinv = lax.rsqrt(deg)
    af_ref[...] = jnp.broadcast_to((1.0 - ALPHA) / deg, af_ref.shape).astype(BF)
    sqdb_ref[...] = jnp.broadcast_to(deg * dinv, sqdb_ref.shape).astype(BF)
    xsd = (ALPHA * dinv * x_ref[...]).astype(BF)
    xsd0_ref[...] = xsd[:, :DH]
    xsd1_ref[...] = xsd[:, DH:]


def kernel(x, edge_index):
    src = edge_index[0]
    dst = edge_index[1]
    src2 = src.reshape(NS, NCH, CH)
    dst2 = dst.reshape(NS, NCH, CH)
    dstf = dst.reshape(NC * NS, EW)
    zdeg = jnp.zeros((N,), jnp.float32)
    ridx = jnp.arange(N, dtype=jnp.int32).reshape(DEG_NCH, DEG_CH)
    zrows = jnp.zeros((DCH, DH), BF)

    deg_call = pl.kernel(
        _deg_body,
        out_type=jax.ShapeDtypeStruct((NC, N), jnp.float32),
        mesh=_sc_mesh(),
        scratch_types=[
            pltpu.VMEM((EW,), jnp.int32),
            pltpu.VMEM((N,), jnp.float32),
            pltpu.VMEM((DEG_NCH, DEG_CH), jnp.int32),
            pltpu.VMEM_SHARED((N,), jnp.float32),
        ],
        compiler_params=pltpu.CompilerParams(needs_layout_passes=False),
    )
    deg_parts = deg_call(dstf, zdeg, ridx)
    degp = jnp.transpose(deg_parts)

    prep_call = pl.pallas_call(
        _prep_body,
        grid=(N // _RB,),
        in_specs=[
            pl.BlockSpec((_RB, 2), lambda i: (i, 0)),
            pl.BlockSpec((_RB, D), lambda i: (i, 0)),
        ],
        out_specs=[
            pl.BlockSpec((_RB, DH), lambda i: (i, 0)),
            pl.BlockSpec((_RB, DH), lambda i: (i, 0)),
            pl.BlockSpec((_RB, DH), lambda i: (i, 0)),
            pl.BlockSpec((_RB, DH), lambda i: (i, 0)),
        ],
        out_shape=[
            jax.ShapeDtypeStruct((NPAD, DH), BF),
            jax.ShapeDtypeStruct((NPAD, DH), BF),
            jax.ShapeDtypeStruct((NPAD, DH), BF),
            jax.ShapeDtypeStruct((NPAD, DH), BF),
        ],
    )
    afp, sqdb, xsd0p, xsd1p = prep_call(degp, x)

    mono_call = pl.kernel(
        _mono_body,
        out_type=[
            jax.ShapeDtypeStruct((NPAD, DH), BF),
            jax.ShapeDtypeStruct((NPAD, DH), BF),
            jax.ShapeDtypeStruct((NPAD, DH), BF),
            jax.ShapeDtypeStruct((NPAD, DH), BF),
        ],
        mesh=_sc_mesh(),
        scratch_types=[
            pltpu.VMEM((NCH, CH), jnp.int32),
            pltpu.VMEM((NCH, CH), jnp.int32),
            pltpu.VMEM((CH, DH), BF),
            pltpu.VMEM((CH, DH), BF),
            pltpu.VMEM((CH, DH), BF),
            pltpu.VMEM((CH, DH), BF),
            pltpu.VMEM((CH, DH), BF),
            pltpu.VMEM((RPS, DH), BF),
            pltpu.VMEM((DCH, DH), BF),
            pltpu.VMEM((DCH, DH), BF),
            pltpu.VMEM((RPS, DH), BF),
            pltpu.VMEM_SHARED((NPAD, DH), BF),
            pltpu.SemaphoreType.DMA,
            pltpu.SemaphoreType.DMA,
            pltpu.SemaphoreType.DMA,
            pltpu.SemaphoreType.DMA,
            pltpu.SemaphoreType.DMA,
        ],
        compiler_params=pltpu.CompilerParams(needs_layout_passes=False,
                                             use_tc_tiling_on_sc=False),
    )
    oout0, oout1, _, _ = mono_call(xsd0p, xsd1p, afp, sqdb, src2, dst2,
                                   zrows)

    return jnp.concatenate([oout0[:N], oout1[:N]],
                           axis=1).astype(jnp.float32)


# speedup vs baseline: 1.0280x; 1.0280x over previous
import jax
import jax.numpy as jnp
from jax import lax
from jax.experimental import pallas as pl
from jax.experimental.pallas import tpu as pltpu
from jax.experimental.pallas import tpu_sc as plsc

N = 10000
E = 320000
D = 128
DH = D // 2
K = 5
ALPHA = 0.8

NC = 2
NS = 16
ES = E // NS
CH = 125
NCH = ES // CH
NPAD = 10240
RPS = NPAD // NS
DCH = 128
NDCH = RPS // DCH
NBUF = 5

DEG_CH = 80
DEG_NCH = N // DEG_CH
EW = E // (NC * NS)

BF = jnp.bfloat16


def _deg_body(dst_hbm, zdeg_hbm, ridx_hbm, deg_out_hbm,
              dst_v, deg_l, ridx_v, deg_sp):
    c = lax.axis_index("c")
    s = lax.axis_index("s")
    w = s * NC + c

    pltpu.sync_copy(zdeg_hbm, deg_l)

    @pl.when(s == 0)
    def _():
        pltpu.sync_copy(deg_l, deg_sp)

    plsc.subcore_barrier()

    pltpu.sync_copy(dst_hbm.at[w], dst_v)
    pltpu.sync_copy(ridx_hbm, ridx_v)
    ones = jnp.ones((16,), jnp.float32)

    @pl.loop(0, EW // 16)
    def _(i):
        d = dst_v[pl.ds(i * 16, 16)]
        plsc.addupdate_scatter(deg_l, [d], ones)

    @pl.loop(0, DEG_NCH)
    def _(j):
        pltpu.sync_copy(deg_l.at[pl.ds(j * DEG_CH, DEG_CH)],
                        deg_sp.at[ridx_v.at[j]], add=True)

    plsc.subcore_barrier()

    @pl.when(s == 0)
    def _():
        pltpu.sync_copy(deg_sp, deg_l)
        pltpu.sync_copy(deg_l, deg_out_hbm.at[c])


def _mono_body(xsd0_hbm, xsd1_hbm, af_hbm, src_hbm, dst_hbm, zero_hbm,
               oout0_hbm, oout1_hbm, osb0_hbm, osb1_hbm,
               src_v, dst_v, rows0, rows1, rows2, rows3, rows4,
               os_res, xch_v, aggch_v, af_v, agg_sp,
               sg0, sg1, sg2, sg3, sg4):
    rows = (rows0, rows1, rows2, rows3, rows4)
    sg = (sg0, sg1, sg2, sg3, sg4)
    c = lax.axis_index("c")
    s = lax.axis_index("s")
    base = s * RPS

    pltpu.sync_copy(src_hbm.at[s], src_v)
    pltpu.sync_copy(dst_hbm.at[s], dst_v)
    pltpu.sync_copy(af_hbm.at[pl.ds(base, RPS)], af_v)

    def half(xsd_hbm, osb_hbm, oout_hbm):
        @pl.loop(0, NDCH)
        def _(t):
            pltpu.sync_copy(xsd_hbm.at[pl.ds(base + t * DCH, DCH)], xch_v)

            @pl.loop(0, DCH)
            def _(r):
                rr = t * DCH + r
                for g in range(2):
                    sl = pl.ds(32 * g, 32)
                    os_res[rr, sl] = xch_v[r, sl] * jnp.asarray(1.25, BF)

        pltpu.sync_copy(os_res, osb_hbm.at[pl.ds(base, RPS)])

        @pl.loop(0, K)
        def _(k):
            pltpu.sync_copy(zero_hbm, aggch_v)

            @pl.loop(0, NDCH)
            def _(t):
                pltpu.sync_copy(aggch_v, agg_sp.at[pl.ds(base + t * DCH, DCH)])

            plsc.subcore_barrier()

            for b in range(NBUF):
                pltpu.async_copy(osb_hbm.at[src_v.at[b]], rows[b], sg[b])

            @pl.loop(0, NCH // NBUF)
            def _(t):
                for b in range(NBUF):
                    j = t * NBUF + b
                    pltpu.make_async_copy(osb_hbm.at[src_v.at[j]],
                                          rows[b], sg[b]).wait()
                    pltpu.sync_copy(rows[b], agg_sp.at[dst_v.at[j]], add=True)

                    @pl.when(j < NCH - NBUF)
                    def _():
                        pltpu.async_copy(osb_hbm.at[src_v.at[j + NBUF]],
                                         rows[b], sg[b])

            plsc.subcore_barrier()

            @pl.loop(0, NDCH)
            def _(t):
                row0 = base + t * DCH
                pltpu.sync_copy(agg_sp.at[pl.ds(row0, DCH)], aggch_v)
                pltpu.sync_copy(xsd_hbm.at[pl.ds(row0, DCH)], xch_v)

                @pl.loop(0, DCH)
                def _(r):
                    rr = t * DCH + r
                    for g in range(2):
                        sl = pl.ds(32 * g, 32)
                        os_res[rr, sl] = (af_v[rr, sl]
                                          * (aggch_v[r, sl] + os_res[rr, sl])
                                          + xch_v[r, sl])

            pltpu.sync_copy(os_res, osb_hbm.at[pl.ds(base, RPS)])

        pltpu.sync_copy(os_res, oout_hbm.at[pl.ds(base, RPS)])

    @pl.when(c == 0)
    def _():
        half(xsd0_hbm, osb0_hbm, oout0_hbm)

    @pl.when(c == 1)
    def _():
        half(xsd1_hbm, osb1_hbm, oout1_hbm)


def _sc_mesh():
    return plsc.VectorSubcoreMesh(core_axis_name="c", subcore_axis_name="s",
                                  num_cores=NC, num_subcores=NS)


_RB = 2000


def _prep_body(degp_ref, x_ref, af_ref, sqd_ref, xsd0_ref, xsd1_ref):
    p = degp_ref[...]
    deg = p[:, 0:1] + p[:, 1:2] + 1.0
    dinv = lax.rsqrt(deg)
    af_ref[...] = jnp.broadcast_to((1.0 - ALPHA) / deg, af_ref.shape).astype(BF)
    sqd_ref[...] = deg * dinv
    xsd = (ALPHA * dinv * x_ref[...]).astype(BF)
    xsd0_ref[...] = xsd[:, :DH]
    xsd1_ref[...] = xsd[:, DH:]


def _post_body(o0_ref, o1_ref, sqd_ref, out_ref):
    os_full = jnp.concatenate([o0_ref[...], o1_ref[...]],
                              axis=1).astype(jnp.float32)
    out_ref[...] = os_full * sqd_ref[...]


def kernel(x, edge_index):
    src = edge_index[0]
    dst = edge_index[1]
    src2 = src.reshape(NS, NCH, CH)
    dst2 = dst.reshape(NS, NCH, CH)
    dstf = dst.reshape(NC * NS, EW)
    zdeg = jnp.zeros((N,), jnp.float32)
    ridx = jnp.arange(N, dtype=jnp.int32).reshape(DEG_NCH, DEG_CH)
    zrows = jnp.zeros((DCH, DH), BF)

    deg_call = pl.kernel(
        _deg_body,
        out_type=jax.ShapeDtypeStruct((NC, N), jnp.float32),
        mesh=_sc_mesh(),
        scratch_types=[
            pltpu.VMEM((EW,), jnp.int32),
            pltpu.VMEM((N,), jnp.float32),
            pltpu.VMEM((DEG_NCH, DEG_CH), jnp.int32),
            pltpu.VMEM_SHARED((N,), jnp.float32),
        ],
        compiler_params=pltpu.CompilerParams(needs_layout_passes=False),
    )
    deg_parts = deg_call(dstf, zdeg, ridx)
    degp = jnp.transpose(deg_parts)

    prep_call = pl.pallas_call(
        _prep_body,
        grid=(N // _RB,),
        in_specs=[
            pl.BlockSpec((_RB, 2), lambda i: (i, 0)),
            pl.BlockSpec((_RB, D), lambda i: (i, 0)),
        ],
        out_specs=[
            pl.BlockSpec((_RB, DH), lambda i: (i, 0)),
            pl.BlockSpec((_RB, 1), lambda i: (i, 0)),
            pl.BlockSpec((_RB, DH), lambda i: (i, 0)),
            pl.BlockSpec((_RB, DH), lambda i: (i, 0)),
        ],
        out_shape=[
            jax.ShapeDtypeStruct((NPAD, DH), BF),
            jax.ShapeDtypeStruct((N, 1), jnp.float32),
            jax.ShapeDtypeStruct((NPAD, DH), BF),
            jax.ShapeDtypeStruct((NPAD, DH), BF),
        ],
    )
    afp, sqd, xsd0p, xsd1p = prep_call(degp, x)

    mono_call = pl.kernel(
        _mono_body,
        out_type=[
            jax.ShapeDtypeStruct((NPAD, DH), BF),
            jax.ShapeDtypeStruct((NPAD, DH), BF),
            jax.ShapeDtypeStruct((NPAD, DH), BF),
            jax.ShapeDtypeStruct((NPAD, DH), BF),
        ],
        mesh=_sc_mesh(),
        scratch_types=[
            pltpu.VMEM((NCH, CH), jnp.int32),
            pltpu.VMEM((NCH, CH), jnp.int32),
            pltpu.VMEM((CH, DH), BF),
            pltpu.VMEM((CH, DH), BF),
            pltpu.VMEM((CH, DH), BF),
            pltpu.VMEM((CH, DH), BF),
            pltpu.VMEM((CH, DH), BF),
            pltpu.VMEM((RPS, DH), BF),
            pltpu.VMEM((DCH, DH), BF),
            pltpu.VMEM((DCH, DH), BF),
            pltpu.VMEM((RPS, DH), BF),
            pltpu.VMEM_SHARED((NPAD, DH), BF),
            pltpu.SemaphoreType.DMA,
            pltpu.SemaphoreType.DMA,
            pltpu.SemaphoreType.DMA,
            pltpu.SemaphoreType.DMA,
            pltpu.SemaphoreType.DMA,
        ],
        compiler_params=pltpu.CompilerParams(needs_layout_passes=False,
                                             use_tc_tiling_on_sc=False),
    )
    oout0, oout1, _, _ = mono_call(xsd0p, xsd1p, afp, src2, dst2, zrows)

    post_call = pl.pallas_call(
        _post_body,
        grid=(N // _RB,),
        in_specs=[
            pl.BlockSpec((_RB, DH), lambda i: (i, 0)),
            pl.BlockSpec((_RB, DH), lambda i: (i, 0)),
            pl.BlockSpec((_RB, 1), lambda i: (i, 0)),
        ],
        out_specs=pl.BlockSpec((_RB, D), lambda i: (i, 0)),
        out_shape=jax.ShapeDtypeStruct((N, D), jnp.float32),
    )
    return post_call(oout0, oout1, sqd)
